# baseline (speedup 1.0000x reference)
import functools

import jax
import jax.numpy as jnp
from jax import lax
from jax.experimental import pallas as pl
from jax.experimental.pallas import tpu as pltpu
from jax.experimental.pallas import tpu_sc as plsc

B = 16384
D = 16
NROWS = 1000000
RCHUNK = 2048
RGRID = (NROWS + RCHUNK - 1) // RCHUNK
CHUNK = 128


def _reformat_body(xug, xig, xum, xim, oug, oig, oum, oim):
  for x, o in ((xug, oug), (xig, oig), (xum, oum), (xim, oim)):
    o[...] = jnp.transpose(x[...])


@jax.jit
def _tc_reformat(tug, tig, tum, tim):
  tbl = jax.ShapeDtypeStruct((NROWS, D), jnp.float32)
  in_spec = pl.BlockSpec((D, RCHUNK), lambda c: (0, c))
  out_spec = pl.BlockSpec((RCHUNK, D), lambda c: (c, 0))
  return pl.pallas_call(
      _reformat_body,
      grid=(RGRID,),
      in_specs=[in_spec] * 4,
      out_specs=[out_spec] * 4,
      out_shape=(tbl, tbl, tbl, tbl),
  )(tug, tig, tum, tim)


def _sc_gather_body(nc, ns, bpw,
                    uid, iid, tug, tig, tum, tim,
                    oug, oig, oum, oim,
                    xu, xi, bug, big, bum, bim, sem):
  wid = lax.axis_index("s") * nc + lax.axis_index("c")
  base = wid * bpw
  pltpu.sync_copy(uid.at[pl.ds(base, bpw)], xu)
  pltpu.sync_copy(iid.at[pl.ds(base, bpw)], xi)
  copies = []
  for j in range(bpw // CHUNK):
    sl = pl.ds(j * CHUNK, CHUNK)
    copies.append(pltpu.async_copy(tug.at[xu.at[sl]], bug.at[sl], sem))
    copies.append(pltpu.async_copy(tig.at[xi.at[sl]], big.at[sl], sem))
    copies.append(pltpu.async_copy(tum.at[xu.at[sl]], bum.at[sl], sem))
    copies.append(pltpu.async_copy(tim.at[xi.at[sl]], bim.at[sl], sem))
  for c in copies:
    c.wait()
  pltpu.sync_copy(bug, oug.at[pl.ds(base, bpw)])
  pltpu.sync_copy(big, oig.at[pl.ds(base, bpw)])
  pltpu.sync_copy(bum, oum.at[pl.ds(base, bpw)])
  pltpu.sync_copy(bim, oim.at[pl.ds(base, bpw)])


@jax.jit
def _sc_gather(uid, iid, tug, tig, tum, tim):
  info = plsc.get_sparse_core_info()
  nc, ns = info.num_cores, info.num_subcores
  nw = nc * ns
  bpw = B // nw
  mesh = plsc.VectorSubcoreMesh(core_axis_name="c", subcore_axis_name="s")
  row = jax.ShapeDtypeStruct((B, D), jnp.float32)
  body = functools.partial(_sc_gather_body, nc, ns, bpw)
  return pl.kernel(
      body,
      mesh=mesh,
      compiler_params=pltpu.CompilerParams(use_tc_tiling_on_sc=False),
      out_type=(row, row, row, row),
      scratch_types=[
          pltpu.VMEM((bpw,), jnp.int32),
          pltpu.VMEM((bpw,), jnp.int32),
          pltpu.VMEM((bpw, D), jnp.float32),
          pltpu.VMEM((bpw, D), jnp.float32),
          pltpu.VMEM((bpw, D), jnp.float32),
          pltpu.VMEM((bpw, D), jnp.float32),
          pltpu.SemaphoreType.DMA,
      ],
  )(uid, iid, tug, tig, tum, tim)


def _bn_relu(x, g, be):
  mean = jnp.mean(x, axis=0)
  var = jnp.mean((x - mean) ** 2, axis=0)
  x = (x - mean) * lax.rsqrt(var + 1e-5) * g + be
  return jnp.maximum(x, 0.0)


def _tc_body(ug, ig, um, im,
             W0, b0, g0, be0, W1, b1, g1, be1,
             W2, b2, g2, be2, W3, b3, g3, be3,
             Wp, bp, out):
  f32 = jnp.float32
  x = (jnp.dot(um[...], W0[0:D, :], preferred_element_type=f32)
       + jnp.dot(im[...], W0[D:2 * D, :], preferred_element_type=f32)
       + b0[...])
  x = _bn_relu(x, g0[...], be0[...])
  x = jnp.dot(x, W1[...], preferred_element_type=f32) + b1[...]
  x = _bn_relu(x, g1[...], be1[...])
  x = jnp.dot(x, W2[...], preferred_element_type=f32) + b2[...]
  x = _bn_relu(x, g2[...], be2[...])
  x = jnp.dot(x, W3[...], preferred_element_type=f32) + b3[...]
  x = _bn_relu(x, g3[...], be3[...])
  gmf = ug[...] * ig[...]
  logit = (jnp.dot(gmf, Wp[0:D, :], preferred_element_type=f32)
           + jnp.dot(x, Wp[D:D + 8, :], preferred_element_type=f32)
           + bp[...])
  out[...] = jax.nn.sigmoid(logit)


@jax.jit
def _tc_mlp(ug, ig, um, im, *weights):
  return pl.pallas_call(
      _tc_body,
      out_shape=jax.ShapeDtypeStruct((B, 1), jnp.float32),
  )(ug, ig, um, im, *weights)


def kernel(user_indices, item_indices, user_gmf, item_gmf, user_mlp, item_mlp,
           W0, b0, g0, be0, W1, b1, g1, be1, W2, b2, g2, be2, W3, b3, g3, be3,
           Wp, bp):
  uid = user_indices.astype(jnp.int32)
  iid = item_indices.astype(jnp.int32)
  rug, rig, rum, rim = _tc_reformat(user_gmf.T, item_gmf.T,
                                    user_mlp.T, item_mlp.T)
  ug, ig, um, im = _sc_gather(uid, iid, rug, rig, rum, rim)
  pred = _tc_mlp(ug, ig, um, im,
                 W0, b0, g0, be0, W1, b1, g1, be1,
                 W2, b2, g2, be2, W3, b3, g3, be3, Wp, bp)
  return jnp.squeeze(pred, axis=-1)


# speedup vs baseline: 1.8186x; 1.8186x over previous
import functools

import jax
import jax.numpy as jnp
from jax import lax
from jax.experimental import pallas as pl
from jax.experimental.pallas import tpu as pltpu
from jax.experimental.pallas import tpu_sc as plsc

B = 16384
D = 16
NROWS = 1000000
PACK = 8
SUPER = 128 * PACK
PROWS = 125056
RCH = 1024
RGRID = 123
GCHUNK = 128


def _reformat_body(xug, xig, xum, xim, oug, oig, oum, oim):
  for x, o in ((xug, oug), (xig, oig), (xum, oum), (xim, oim)):
    rows = []
    for sb in range(PACK):
      rows.append(jnp.concatenate(
          [jnp.transpose(x[:, 1024 * sb + 128 * s:1024 * sb + 128 * s + 128])
           for s in range(PACK)], axis=1))
    o[...] = jnp.concatenate(rows, axis=0)


@jax.jit
def _tc_reformat(tug, tig, tum, tim):
  tbl = jax.ShapeDtypeStruct((PROWS, PACK * D), jnp.float32)
  in_spec = pl.BlockSpec((D, PACK * SUPER), lambda c: (0, c))
  out_spec = pl.BlockSpec((RCH, PACK * D), lambda c: (c, 0))
  return pl.pallas_call(
      _reformat_body,
      grid=(RGRID,),
      in_specs=[in_spec] * 4,
      out_specs=[out_spec] * 4,
      out_shape=(tbl, tbl, tbl, tbl),
  )(tug, tig, tum, tim)


def _sc_gather_body(nc, ns, bpw,
                    uid, iid, tug, tig, tum, tim,
                    oug, oig, oum, oim,
                    xu, xi, bug, big, bum, bim, sem):
  wid = lax.axis_index("s") * nc + lax.axis_index("c")
  base = wid * bpw
  pltpu.sync_copy(uid.at[pl.ds(base, bpw)], xu)
  pltpu.sync_copy(iid.at[pl.ds(base, bpw)], xi)
  for k in range(bpw // D):
    sl = pl.ds(k * D, D)
    for x in (xu, xi):
      v = x[sl]
      x[sl] = jnp.bitwise_or(
          lax.shift_left(lax.shift_right_logical(v, 10), 7),
          jnp.bitwise_and(v, 127))
  for k in range(bpw // GCHUNK):
    sl = pl.ds(k * GCHUNK, GCHUNK)
    copies = [
        pltpu.async_copy(tug.at[xu.at[sl]], bug, sem),
        pltpu.async_copy(tig.at[xi.at[sl]], big, sem),
        pltpu.async_copy(tum.at[xu.at[sl]], bum, sem),
        pltpu.async_copy(tim.at[xi.at[sl]], bim, sem),
    ]
    for c in copies:
      c.wait()
    out_sl = pl.ds(base + k * GCHUNK, GCHUNK)
    pltpu.sync_copy(bug, oug.at[out_sl])
    pltpu.sync_copy(big, oig.at[out_sl])
    pltpu.sync_copy(bum, oum.at[out_sl])
    pltpu.sync_copy(bim, oim.at[out_sl])


@jax.jit
def _sc_gather(uid, iid, tug, tig, tum, tim):
  info = plsc.get_sparse_core_info()
  nc, ns = info.num_cores, info.num_subcores
  nw = nc * ns
  bpw = B // nw
  mesh = plsc.VectorSubcoreMesh(core_axis_name="c", subcore_axis_name="s")
  out = jax.ShapeDtypeStruct((B, PACK * D), jnp.float32)
  body = functools.partial(_sc_gather_body, nc, ns, bpw)
  return pl.kernel(
      body,
      mesh=mesh,
      compiler_params=pltpu.CompilerParams(use_tc_tiling_on_sc=False),
      out_type=(out, out, out, out),
      scratch_types=[
          pltpu.VMEM((bpw,), jnp.int32),
          pltpu.VMEM((bpw,), jnp.int32),
          pltpu.VMEM((GCHUNK, PACK * D), jnp.float32),
          pltpu.VMEM((GCHUNK, PACK * D), jnp.float32),
          pltpu.VMEM((GCHUNK, PACK * D), jnp.float32),
          pltpu.VMEM((GCHUNK, PACK * D), jnp.float32),
          pltpu.SemaphoreType.DMA,
      ],
  )(uid, iid, tug, tig, tum, tim)


HCH = 1024


def _extract(packed, sub):
  acc = jnp.zeros((HCH, D), jnp.float32)
  for s in range(PACK):
    acc = acc + jnp.where(sub == s, packed[:, s * D:(s + 1) * D], 0.0)
  return acc


def _head_body(pug, pig, pum, pim, uid2, iid2, W0, b0, out):
  f32 = jnp.float32
  su = jnp.bitwise_and(lax.shift_right_logical(uid2[...], 7), 7)
  si = jnp.bitwise_and(lax.shift_right_logical(iid2[...], 7), 7)
  ug = _extract(pug[...], su)
  ig = _extract(pig[...], si)
  um = _extract(pum[...], su)
  im = _extract(pim[...], si)
  h0 = (jnp.dot(um, W0[0:D, :], preferred_element_type=f32)
        + jnp.dot(im, W0[D:2 * D, :], preferred_element_type=f32)
        + b0[...])
  gmf = ug * ig
  out[...] = jnp.concatenate(
      [h0, gmf, jnp.zeros((HCH, 128 - 64 - D), f32)], axis=1)


@jax.jit
def _tc_head(pug, pig, pum, pim, uid2, iid2, W0, b0):
  bspec = pl.BlockSpec((HCH, PACK * D), lambda c: (c, 0))
  ispec = pl.BlockSpec((HCH, 1), lambda c: (c, 0))
  wspec = pl.BlockSpec((2 * D, 64), lambda c: (0, 0))
  b0spec = pl.BlockSpec((64,), lambda c: (0,))
  return pl.pallas_call(
      _head_body,
      grid=(B // HCH,),
      in_specs=[bspec] * 4 + [ispec] * 2 + [wspec, b0spec],
      out_specs=pl.BlockSpec((HCH, PACK * D), lambda c: (c, 0)),
      out_shape=jax.ShapeDtypeStruct((B, PACK * D), jnp.float32),
  )(pug, pig, pum, pim, uid2, iid2, W0, b0)


def _bn_relu(x, g, be):
  mean = jnp.mean(x, axis=0)
  var = jnp.mean((x - mean) ** 2, axis=0)
  x = (x - mean) * lax.rsqrt(var + 1e-5) * g + be
  return jnp.maximum(x, 0.0)


def _tail_body(hg, g0, be0, W1, b1, g1, be1,
               W2, b2, g2, be2, W3, b3, g3, be3,
               Wp, bp, out):
  f32 = jnp.float32
  x = _bn_relu(hg[:, 0:64], g0[...], be0[...])
  gmf = hg[:, 64:64 + D]
  x = jnp.dot(x, W1[...], preferred_element_type=f32) + b1[...]
  x = _bn_relu(x, g1[...], be1[...])
  x = jnp.dot(x, W2[...], preferred_element_type=f32) + b2[...]
  x = _bn_relu(x, g2[...], be2[...])
  x = jnp.dot(x, W3[...], preferred_element_type=f32) + b3[...]
  x = _bn_relu(x, g3[...], be3[...])
  logit = (jnp.dot(gmf, Wp[0:D, :], preferred_element_type=f32)
           + jnp.dot(x, Wp[D:D + 8, :], preferred_element_type=f32)
           + bp[...])
  out[...] = jax.nn.sigmoid(logit)


@jax.jit
def _tc_tail(hg, *weights):
  return pl.pallas_call(
      _tail_body,
      out_shape=jax.ShapeDtypeStruct((B, 1), jnp.float32),
  )(hg, *weights)


def kernel(user_indices, item_indices, user_gmf, item_gmf, user_mlp, item_mlp,
           W0, b0, g0, be0, W1, b1, g1, be1, W2, b2, g2, be2, W3, b3, g3, be3,
           Wp, bp):
  uid = user_indices.astype(jnp.int32)
  iid = item_indices.astype(jnp.int32)
  rug, rig, rum, rim = _tc_reformat(user_gmf.T, item_gmf.T,
                                    user_mlp.T, item_mlp.T)
  pug, pig, pum, pim = _sc_gather(uid, iid, rug, rig, rum, rim)
  hg = _tc_head(pug, pig, pum, pim,
                uid.reshape(B, 1), iid.reshape(B, 1), W0, b0)
  pred = _tc_tail(hg, g0, be0, W1, b1, g1, be1,
                  W2, b2, g2, be2, W3, b3, g3, be3, Wp, bp)
  return jnp.squeeze(pred, axis=-1)
